# Initial kernel scaffold; baseline (speedup 1.0000x reference)
import dataclasses
import functools

import jax
import jax.numpy as jnp
from jax import lax
from jax.experimental import pallas as pl
from jax.experimental.pallas import tpu as pltpu
from jax.experimental.pallas import tpu_sc as plsc

N = 10000
E = 320000
NP = 10240
NS = 16
CHUNK = 128
EPT = 20480
EP = EPT * NS
NCH = EPT // CHUNK
RPT = NP // NS

_f32 = jnp.float32
_i32 = jnp.int32


def _sc_compiler_params():
    cp = pltpu.CompilerParams()
    if "needs_layout_passes" in pltpu.CompilerParams.__dataclass_fields__:
        cp = dataclasses.replace(cp, needs_layout_passes=False)
    return cp


_MESH = plsc.VectorSubcoreMesh(core_axis_name="c", subcore_axis_name="s")


@functools.partial(
    pl.kernel,
    out_type=jax.ShapeDtypeStruct((2, NP, 16), _f32),
    mesh=_MESH,
    scratch_types=[
        pltpu.VMEM((NCH, CHUNK), _i32),
        pltpu.VMEM((NCH, CHUNK), _f32),
        pltpu.VMEM((CHUNK, 16), _f32),
        pltpu.VMEM_SHARED((NP, 16), _f32),
    ],
    compiler_params=_sc_compiler_params(),
)
def _deg_kernel(dstr, ewr, out, dst_v, ew_v, vals_v, acc):
    c = lax.axis_index("c")
    s = lax.axis_index("s")
    pltpu.sync_copy(dstr.at[s], dst_v)
    pltpu.sync_copy(ewr.at[s], ew_v)

    @pl.loop(0, CHUNK)
    def _(r):
        vals_v[r, pl.ds(0, 16)] = jnp.zeros((16,), _f32)

    for m in range(RPT // CHUNK):
        pltpu.sync_copy(vals_v, acc.at[pl.ds(s * RPT + m * CHUNK, CHUNK)])
    plsc.subcore_barrier()

    half = NCH // 2

    @pl.loop(0, half)
    def _(j0):
        j = j0 + c * half
        for k in range(CHUNK // 16):
            sl = pl.ds(k * 16, 16)
            ridx = lax.iota(_i32, 16) + (k * 16)
            plsc.store_scatter(vals_v, [ridx, jnp.zeros((16,), _i32)],
                               ew_v[j, sl])
        pltpu.sync_copy(vals_v, acc.at[dst_v.at[j]], add=True)

    plsc.subcore_barrier()
    pltpu.sync_copy(acc.at[pl.ds(s * RPT, RPT)], out.at[c, pl.ds(s * RPT, RPT)])


@functools.partial(
    pl.kernel,
    out_type=(jax.ShapeDtypeStruct((NP, 128), _f32),
              jax.ShapeDtypeStruct((NP, 128), _f32)),
    mesh=_MESH,
    scratch_types=[
        pltpu.VMEM((NCH, CHUNK), _i32),
        pltpu.VMEM((NCH, CHUNK), _i32),
        pltpu.VMEM((NCH, CHUNK), _f32),
        pltpu.VMEM((NP,), _f32),
        pltpu.VMEM((CHUNK,), _f32),
        pltpu.VMEM((CHUNK, 128), _f32),
        pltpu.VMEM_SHARED((NP, 128), _f32),
    ],
    compiler_params=_sc_compiler_params(),
)
def _conv_kernel(t0, t1, srcr, dstr, ewr, disr, out0, out1,
                 src_v, dst_v, ew_v, dis_v, sval_v, rows_v, acc):
    c = lax.axis_index("c")
    s = lax.axis_index("s")
    pltpu.sync_copy(srcr.at[s], src_v)
    pltpu.sync_copy(dstr.at[s], dst_v)
    pltpu.sync_copy(ewr.at[s], ew_v)
    pltpu.sync_copy(disr, dis_v)

    @pl.loop(0, CHUNK)
    def _(r):
        for k in range(8):
            rows_v[r, pl.ds(k * 16, 16)] = jnp.zeros((16,), _f32)

    for m in range(RPT // CHUNK):
        pltpu.sync_copy(rows_v, acc.at[pl.ds(s * RPT + m * CHUNK, CHUNK)])
    plsc.subcore_barrier()

    @pl.loop(0, NCH)
    def _(j):
        @pl.when(c == 0)
        def _():
            pltpu.sync_copy(t0.at[src_v.at[j]], rows_v)

        @pl.when(c == 1)
        def _():
            pltpu.sync_copy(t1.at[src_v.at[j]], rows_v)

        for k in range(CHUNK // 16):
            sl = pl.ds(k * 16, 16)
            dsk = plsc.load_gather(dis_v, [src_v[j, sl]])
            sval_v[sl] = ew_v[j, sl] * dsk

        @pl.loop(0, CHUNK)
        def _(r):
            sv = plsc.load_gather(sval_v, [jnp.full((16,), r, _i32)])
            for k in range(8):
                sl = pl.ds(k * 16, 16)
                rows_v[r, sl] = rows_v[r, sl] * sv

        pltpu.sync_copy(rows_v, acc.at[dst_v.at[j]], add=True)

    plsc.subcore_barrier()

    @pl.when(c == 0)
    def _():
        pltpu.sync_copy(acc.at[pl.ds(s * RPT, RPT)],
                        out0.at[pl.ds(s * RPT, RPT)])

    @pl.when(c == 1)
    def _():
        pltpu.sync_copy(acc.at[pl.ds(s * RPT, RPT)],
                        out1.at[pl.ds(s * RPT, RPT)])


_BN = 1000
_BE = 4000


def _dot(a, b):
    return jnp.dot(a, b, preferred_element_type=_f32)


def _edge_mlp_body(ef, w1, b1, w2, b2, out):
    hh = jnp.maximum(_dot(ef[...], w1[...]) + b1[...], 0.0)
    out[...] = _dot(hh, w2[...]) + b2[...]


def _edge_mlp(ef, w1, b1, w2, b2):
    return pl.pallas_call(
        _edge_mlp_body,
        grid=(E // _BE,),
        in_specs=[
            pl.BlockSpec((_BE, 16), lambda i: (i, 0)),
            pl.BlockSpec((16, 64), lambda i: (0, 0)),
            pl.BlockSpec((1, 64), lambda i: (0, 0)),
            pl.BlockSpec((64, 1), lambda i: (0, 0)),
            pl.BlockSpec((1, 1), lambda i: (0, 0)),
        ],
        out_specs=pl.BlockSpec((_BE, 1), lambda i: (i, 0)),
        out_shape=jax.ShapeDtypeStruct((E, 1), _f32),
    )(ef, w1, b1, w2, b2)


def _node_mlp_body(x, wn1, bn1, wn2, bn2, wg1, ta, tb):
    hh = jnp.maximum(_dot(x[...], wn1[...]) + bn1[...], 0.0)
    hh = _dot(hh, wn2[...]) + bn2[...]
    t = _dot(hh, wg1[...])
    ta[...] = t[:, :128]
    tb[...] = t[:, 128:]


def _node_mlp(x, wn1, bn1, wn2, bn2, wg1):
    return pl.pallas_call(
        _node_mlp_body,
        grid=(N // _BN,),
        in_specs=[
            pl.BlockSpec((_BN, 128), lambda i: (i, 0)),
            pl.BlockSpec((128, 256), lambda i: (0, 0)),
            pl.BlockSpec((1, 256), lambda i: (0, 0)),
            pl.BlockSpec((256, 256), lambda i: (0, 0)),
            pl.BlockSpec((1, 256), lambda i: (0, 0)),
            pl.BlockSpec((256, 256), lambda i: (0, 0)),
        ],
        out_specs=(pl.BlockSpec((_BN, 128), lambda i: (i, 0)),
                   pl.BlockSpec((_BN, 128), lambda i: (i, 0))),
        out_shape=(jax.ShapeDtypeStruct((N, 128), _f32),
                   jax.ShapeDtypeStruct((N, 128), _f32)),
    )(x, wn1, bn1, wn2, bn2, wg1)


def _dis_body(degs, dis):
    a = degs[...]
    deg = a[0, :, 0:1] + a[1, :, 0:1] + 1.0
    dis[...] = jnp.where(deg > 0, lax.rsqrt(jnp.maximum(deg, 1e-12)), 0.0)


def _dis_kernel(degs):
    return pl.pallas_call(
        _dis_body,
        out_shape=jax.ShapeDtypeStruct((NP, 1), _f32),
    )(degs)


def _mid_body(agg_a, agg_b, ta, tb, dis, bg1, wg2, oa, ob):
    d = dis[...]
    agg = jnp.concatenate([agg_a[...], agg_b[...]], axis=1)
    t1 = jnp.concatenate([ta[...], tb[...]], axis=1)
    x1 = jnp.maximum(d * (agg + d * t1) + bg1[...], 0.0)
    t2 = _dot(x1, wg2[...])
    oa[...] = t2[:, :128]
    ob[...] = t2[:, 128:]


def _mid_kernel(agg_a, agg_b, ta, tb, dis, bg1, wg2):
    return pl.pallas_call(
        _mid_body,
        grid=(N // _BN,),
        in_specs=[
            pl.BlockSpec((_BN, 128), lambda i: (i, 0)),
            pl.BlockSpec((_BN, 128), lambda i: (i, 0)),
            pl.BlockSpec((_BN, 128), lambda i: (i, 0)),
            pl.BlockSpec((_BN, 128), lambda i: (i, 0)),
            pl.BlockSpec((_BN, 1), lambda i: (i, 0)),
            pl.BlockSpec((1, 256), lambda i: (0, 0)),
            pl.BlockSpec((256, 256), lambda i: (0, 0)),
        ],
        out_specs=(pl.BlockSpec((_BN, 128), lambda i: (i, 0)),
                   pl.BlockSpec((_BN, 128), lambda i: (i, 0))),
        out_shape=(jax.ShapeDtypeStruct((N, 128), _f32),
                   jax.ShapeDtypeStruct((N, 128), _f32)),
    )(agg_a, agg_b, ta, tb, dis, bg1, wg2)


def _final_body(agg_a, agg_b, ta, tb, dis, bg2, wo, bo, out):
    d = dis[...]
    agg = jnp.concatenate([agg_a[...], agg_b[...]], axis=1)
    t2 = jnp.concatenate([ta[...], tb[...]], axis=1)
    x2 = jnp.maximum(d * (agg + d * t2) + bg2[...], 0.0)
    out[...] = _dot(x2, wo[...]) + bo[...]


def _final_kernel(agg_a, agg_b, ta, tb, dis, bg2, wo, bo):
    return pl.pallas_call(
        _final_body,
        grid=(N // _BN,),
        in_specs=[
            pl.BlockSpec((_BN, 128), lambda i: (i, 0)),
            pl.BlockSpec((_BN, 128), lambda i: (i, 0)),
            pl.BlockSpec((_BN, 128), lambda i: (i, 0)),
            pl.BlockSpec((_BN, 128), lambda i: (i, 0)),
            pl.BlockSpec((_BN, 1), lambda i: (i, 0)),
            pl.BlockSpec((1, 256), lambda i: (0, 0)),
            pl.BlockSpec((256, 3), lambda i: (0, 0)),
            pl.BlockSpec((1, 3), lambda i: (0, 0)),
        ],
        out_specs=pl.BlockSpec((_BN, 3), lambda i: (i, 0)),
        out_shape=jax.ShapeDtypeStruct((N, 3), _f32),
    )(agg_a, agg_b, ta, tb, dis, bg2, wo, bo)


def kernel(node_features, edge_index, edge_features,
           W_n1, b_n1, W_n2, b_n2, W_e1, b_e1, W_e2, b_e2,
           W_g1, b_g1, W_g2, b_g2, W_o, b_o):
    src = edge_index[0].astype(_i32)
    dst = edge_index[1].astype(_i32)

    ew = _edge_mlp(edge_features, W_e1, b_e1.reshape(1, 64),
                   W_e2, b_e2.reshape(1, 1)).reshape(E)

    padn = EP - E
    fill = jnp.arange(padn, dtype=_i32)
    src_p = jnp.concatenate([src, fill % N]).reshape(NS, NCH, CHUNK)
    dst_p = jnp.concatenate([dst, fill % NP]).reshape(NS, NCH, CHUNK)
    ew_p = jnp.concatenate([ew, jnp.zeros((padn,), _f32)]).reshape(NS, NCH, CHUNK)

    t1a, t1b = _node_mlp(node_features, W_n1, b_n1.reshape(1, 256),
                         W_n2, b_n2.reshape(1, 256), W_g1)

    degs = _deg_kernel(dst_p, ew_p)
    dis_col = _dis_kernel(degs)
    dis_flat = dis_col.reshape(NP)

    agg1a, agg1b = _conv_kernel(t1a, t1b, src_p, dst_p, ew_p, dis_flat)
    t2a, t2b = _mid_kernel(agg1a[:N], agg1b[:N], t1a, t1b, dis_col[:N],
                           b_g1.reshape(1, 256), W_g2)

    agg2a, agg2b = _conv_kernel(t2a, t2b, src_p, dst_p, ew_p, dis_flat)
    return _final_kernel(agg2a[:N], agg2b[:N], t2a, t2b, dis_col[:N],
                         b_g2.reshape(1, 256), W_o, b_o.reshape(1, 3))


# speedup vs baseline: 7.5625x; 7.5625x over previous
import dataclasses
import functools

import jax
import jax.numpy as jnp
from jax import lax
from jax.experimental import pallas as pl
from jax.experimental.pallas import tpu as pltpu
from jax.experimental.pallas import tpu_sc as plsc

N = 10000
E = 320000
NP = 10240
NS = 16
CHUNK = 128
EPT = 20480
EP = EPT * NS
NCH = EPT // CHUNK
RPT = NP // NS

_f32 = jnp.float32
_i32 = jnp.int32


def _sc_compiler_params():
    cp = pltpu.CompilerParams()
    if "needs_layout_passes" in pltpu.CompilerParams.__dataclass_fields__:
        cp = dataclasses.replace(cp, needs_layout_passes=False)
    return cp


_MESH = plsc.VectorSubcoreMesh(core_axis_name="c", subcore_axis_name="s")


@functools.partial(
    pl.kernel,
    out_type=jax.ShapeDtypeStruct((2 * NP,), _f32),
    mesh=_MESH,
    scratch_types=[
        pltpu.VMEM((8, CHUNK), _i32),
        pltpu.VMEM((8, CHUNK), _f32),
        pltpu.VMEM((CHUNK, 128), _f32),
        pltpu.VMEM((RPT,), _f32),
        pltpu.VMEM_SHARED((NP, 128), _f32),
    ],
    compiler_params=_sc_compiler_params(),
)
def _deg_kernel(dstr, ewr, out, dst_v, ew_v, vals_v, deg1d_v, acc):
    c = lax.axis_index("c")
    s = lax.axis_index("s")

    @pl.loop(0, CHUNK)
    def _(r):
        for k in range(8):
            vals_v[r, pl.ds(k * 16, 16)] = jnp.zeros((16,), _f32)

    for m in range(RPT // CHUNK):
        pltpu.sync_copy(vals_v, acc.at[pl.ds(s * RPT + m * CHUNK, CHUNK)])
    plsc.subcore_barrier()

    @pl.loop(0, NCH // 16)
    def _(g0):
        g = g0 + c * (NCH // 16)
        pltpu.sync_copy(dstr.at[s, pl.ds(g * 8, 8)], dst_v)
        pltpu.sync_copy(ewr.at[s, pl.ds(g * 8, 8)], ew_v)

        for j in range(8):
            for k in range(CHUNK // 16):
                sl = pl.ds(k * 16, 16)
                ridx = lax.iota(_i32, 16) + (k * 16)
                plsc.store_scatter(vals_v, [ridx, jnp.zeros((16,), _i32)],
                                   ew_v[j, sl])
            pltpu.sync_copy(vals_v, acc.at[dst_v.at[j]], add=True)

    plsc.subcore_barrier()

    for m in range(RPT // CHUNK):
        pltpu.sync_copy(acc.at[pl.ds(s * RPT + m * CHUNK, CHUNK)], vals_v)

        @pl.loop(0, CHUNK // 16)
        def _(b):
            ridx = lax.iota(_i32, 16) + b * 16
            v = plsc.load_gather(vals_v, [ridx, jnp.zeros((16,), _i32)])
            deg1d_v[pl.ds(m * CHUNK + b * 16, 16)] = v

    pltpu.sync_copy(deg1d_v, out.at[pl.ds(c * NP + s * RPT, RPT)])


@functools.partial(
    pl.kernel,
    out_type=jax.ShapeDtypeStruct((2 * NP, 128), _f32),
    mesh=_MESH,
    scratch_types=[
        pltpu.VMEM((8, CHUNK), _i32),
        pltpu.VMEM((8, CHUNK), _i32),
        pltpu.VMEM((8, CHUNK), _f32),
        pltpu.VMEM((NP,), _f32),
        pltpu.VMEM((CHUNK,), _f32),
        pltpu.VMEM((CHUNK, 128), _f32),
        pltpu.VMEM_SHARED((NP, 128), _f32),
    ],
    compiler_params=_sc_compiler_params(),
)
def _conv_kernel(t_cat, srcr, dstr, ewr, disr, out, src_v, dst_v, ew_v,
                 dis_v, sval_v, rows_v, acc):
    c = lax.axis_index("c")
    s = lax.axis_index("s")
    pltpu.sync_copy(disr, dis_v)

    @pl.loop(0, CHUNK)
    def _(r):
        for k in range(8):
            rows_v[r, pl.ds(k * 16, 16)] = jnp.zeros((16,), _f32)

    for m in range(RPT // CHUNK):
        pltpu.sync_copy(rows_v, acc.at[pl.ds(s * RPT + m * CHUNK, CHUNK)])
    plsc.subcore_barrier()

    @pl.loop(0, NCH // 8)
    def _(g):
        pltpu.sync_copy(srcr.at[c, s, pl.ds(g * 8, 8)], src_v)
        pltpu.sync_copy(dstr.at[s, pl.ds(g * 8, 8)], dst_v)
        pltpu.sync_copy(ewr.at[s, pl.ds(g * 8, 8)], ew_v)

        for j in range(8):
            pltpu.sync_copy(t_cat.at[src_v.at[j]], rows_v)

            for k in range(CHUNK // 16):
                sl = pl.ds(k * 16, 16)
                srck = src_v[j, sl] - c * N
                dsk = plsc.load_gather(dis_v, [srck])
                sval_v[sl] = ew_v[j, sl] * dsk

            @pl.loop(0, CHUNK)
            def _(r):
                sv = plsc.load_gather(sval_v, [jnp.full((16,), r, _i32)])
                for k in range(8):
                    sl = pl.ds(k * 16, 16)
                    rows_v[r, sl] = rows_v[r, sl] * sv

            pltpu.sync_copy(rows_v, acc.at[dst_v.at[j]], add=True)

    plsc.subcore_barrier()
    pltpu.sync_copy(acc.at[pl.ds(s * RPT, RPT)],
                    out.at[pl.ds(c * NP + s * RPT, RPT)])


_BN = 1000
_BE = 4000


def _dot(a, b):
    return jnp.dot(a, b, preferred_element_type=_f32)


def _edge_mlp_body(ef, w1, b1, w2, b2, out):
    hh = jnp.maximum(_dot(ef[...], w1[...]) + b1[...], 0.0)
    out[...] = _dot(hh, w2[...]) + b2[...]


def _edge_mlp(ef, w1, b1, w2, b2):
    return pl.pallas_call(
        _edge_mlp_body,
        grid=(E // _BE,),
        in_specs=[
            pl.BlockSpec((_BE, 16), lambda i: (i, 0)),
            pl.BlockSpec((16, 64), lambda i: (0, 0)),
            pl.BlockSpec((1, 64), lambda i: (0, 0)),
            pl.BlockSpec((64, 1), lambda i: (0, 0)),
            pl.BlockSpec((1, 1), lambda i: (0, 0)),
        ],
        out_specs=pl.BlockSpec((_BE, 1), lambda i: (i, 0)),
        out_shape=jax.ShapeDtypeStruct((E, 1), _f32),
    )(ef, w1, b1, w2, b2)


def _node_mlp_body(x, wn1, bn1, wn2, bn2, wg1, ta, tb):
    hh = jnp.maximum(_dot(x[...], wn1[...]) + bn1[...], 0.0)
    hh = _dot(hh, wn2[...]) + bn2[...]
    t = _dot(hh, wg1[...])
    ta[...] = t[:, :128]
    tb[...] = t[:, 128:]


def _node_mlp(x, wn1, bn1, wn2, bn2, wg1):
    return pl.pallas_call(
        _node_mlp_body,
        grid=(N // _BN,),
        in_specs=[
            pl.BlockSpec((_BN, 128), lambda i: (i, 0)),
            pl.BlockSpec((128, 256), lambda i: (0, 0)),
            pl.BlockSpec((1, 256), lambda i: (0, 0)),
            pl.BlockSpec((256, 256), lambda i: (0, 0)),
            pl.BlockSpec((1, 256), lambda i: (0, 0)),
            pl.BlockSpec((256, 256), lambda i: (0, 0)),
        ],
        out_specs=(pl.BlockSpec((_BN, 128), lambda i: (i, 0)),
                   pl.BlockSpec((_BN, 128), lambda i: (i, 0))),
        out_shape=(jax.ShapeDtypeStruct((N, 128), _f32),
                   jax.ShapeDtypeStruct((N, 128), _f32)),
    )(x, wn1, bn1, wn2, bn2, wg1)


def _dis_body(degs, dis):
    a = degs[...]
    deg = a[0:1, :] + a[1:2, :] + 1.0
    dis[...] = jnp.where(deg > 0, lax.rsqrt(jnp.maximum(deg, 1e-12)), 0.0)


def _dis_kernel(degs):
    return pl.pallas_call(
        _dis_body,
        out_shape=jax.ShapeDtypeStruct((1, NP), _f32),
    )(degs)


def _mid_body(agg_a, agg_b, ta, tb, dis, bg1, wg2, oa, ob):
    d = dis[...]
    agg = jnp.concatenate([agg_a[...], agg_b[...]], axis=1)
    t1 = jnp.concatenate([ta[...], tb[...]], axis=1)
    x1 = jnp.maximum(d * (agg + d * t1) + bg1[...], 0.0)
    t2 = _dot(x1, wg2[...])
    oa[...] = t2[:, :128]
    ob[...] = t2[:, 128:]


def _mid_kernel(agg_a, agg_b, ta, tb, dis, bg1, wg2):
    return pl.pallas_call(
        _mid_body,
        grid=(N // _BN,),
        in_specs=[
            pl.BlockSpec((_BN, 128), lambda i: (i, 0)),
            pl.BlockSpec((_BN, 128), lambda i: (i, 0)),
            pl.BlockSpec((_BN, 128), lambda i: (i, 0)),
            pl.BlockSpec((_BN, 128), lambda i: (i, 0)),
            pl.BlockSpec((_BN, 1), lambda i: (i, 0)),
            pl.BlockSpec((1, 256), lambda i: (0, 0)),
            pl.BlockSpec((256, 256), lambda i: (0, 0)),
        ],
        out_specs=(pl.BlockSpec((_BN, 128), lambda i: (i, 0)),
                   pl.BlockSpec((_BN, 128), lambda i: (i, 0))),
        out_shape=(jax.ShapeDtypeStruct((N, 128), _f32),
                   jax.ShapeDtypeStruct((N, 128), _f32)),
    )(agg_a, agg_b, ta, tb, dis, bg1, wg2)


def _final_body(agg_a, agg_b, ta, tb, dis, bg2, wo, bo, out):
    d = dis[...]
    agg = jnp.concatenate([agg_a[...], agg_b[...]], axis=1)
    t2 = jnp.concatenate([ta[...], tb[...]], axis=1)
    x2 = jnp.maximum(d * (agg + d * t2) + bg2[...], 0.0)
    out[...] = _dot(x2, wo[...]) + bo[...]


def _final_kernel(agg_a, agg_b, ta, tb, dis, bg2, wo, bo):
    return pl.pallas_call(
        _final_body,
        grid=(N // _BN,),
        in_specs=[
            pl.BlockSpec((_BN, 128), lambda i: (i, 0)),
            pl.BlockSpec((_BN, 128), lambda i: (i, 0)),
            pl.BlockSpec((_BN, 128), lambda i: (i, 0)),
            pl.BlockSpec((_BN, 128), lambda i: (i, 0)),
            pl.BlockSpec((_BN, 1), lambda i: (i, 0)),
            pl.BlockSpec((1, 256), lambda i: (0, 0)),
            pl.BlockSpec((256, 3), lambda i: (0, 0)),
            pl.BlockSpec((1, 3), lambda i: (0, 0)),
        ],
        out_specs=pl.BlockSpec((_BN, 3), lambda i: (i, 0)),
        out_shape=jax.ShapeDtypeStruct((N, 3), _f32),
    )(agg_a, agg_b, ta, tb, dis, bg2, wo, bo)


def kernel(node_features, edge_index, edge_features,
           W_n1, b_n1, W_n2, b_n2, W_e1, b_e1, W_e2, b_e2,
           W_g1, b_g1, W_g2, b_g2, W_o, b_o):
    src = edge_index[0].astype(_i32)
    dst = edge_index[1].astype(_i32)

    ew = _edge_mlp(edge_features, W_e1, b_e1.reshape(1, 64),
                   W_e2, b_e2.reshape(1, 1)).reshape(E)

    padn = EP - E
    fill = jnp.arange(padn, dtype=_i32)
    src_p = jnp.concatenate([src, fill % N]).reshape(NS, NCH, CHUNK)
    src_2 = jnp.stack([src_p, src_p + N])
    dst_p = jnp.concatenate([dst, fill % NP]).reshape(NS, NCH, CHUNK)
    ew_p = jnp.concatenate([ew, jnp.zeros((padn,), _f32)]).reshape(NS, NCH, CHUNK)

    t1a, t1b = _node_mlp(node_features, W_n1, b_n1.reshape(1, 256),
                         W_n2, b_n2.reshape(1, 256), W_g1)

    degs = _deg_kernel(dst_p, ew_p)
    dis_row = _dis_kernel(degs.reshape(2, NP))
    dis_flat = dis_row.reshape(NP)
    dis_col = dis_row.reshape(NP, 1)

    agg1 = _conv_kernel(jnp.concatenate([t1a, t1b], 0), src_2, dst_p, ew_p,
                        dis_flat)
    t2a, t2b = _mid_kernel(agg1[:N], agg1[NP:NP + N], t1a, t1b, dis_col[:N],
                           b_g1.reshape(1, 256), W_g2)

    agg2 = _conv_kernel(jnp.concatenate([t2a, t2b], 0), src_2, dst_p, ew_p,
                        dis_flat)
    return _final_kernel(agg2[:N], agg2[NP:NP + N], t2a, t2b, dis_col[:N],
                         b_g2.reshape(1, 256), W_o, b_o.reshape(1, 3))


# speedup vs baseline: 10.1235x; 1.3386x over previous
import dataclasses
import functools

import jax
import jax.numpy as jnp
from jax import lax
from jax.experimental import pallas as pl
from jax.experimental.pallas import tpu as pltpu
from jax.experimental.pallas import tpu_sc as plsc

N = 10000
E = 320000
NP = 10240
NS = 16
CHUNK = 128
EPT = 20480
EP = EPT * NS
NCH = EPT // CHUNK
RPT = NP // NS

_f32 = jnp.float32
_i32 = jnp.int32


def _sc_compiler_params():
    cp = pltpu.CompilerParams()
    if "needs_layout_passes" in pltpu.CompilerParams.__dataclass_fields__:
        cp = dataclasses.replace(cp, needs_layout_passes=False)
    return cp


_MESH = plsc.VectorSubcoreMesh(core_axis_name="c", subcore_axis_name="s")


@functools.partial(
    pl.kernel,
    out_type=jax.ShapeDtypeStruct((2 * NP,), _f32),
    mesh=_MESH,
    scratch_types=[
        pltpu.VMEM((8, CHUNK), _i32),
        pltpu.VMEM((8, CHUNK), _f32),
        pltpu.VMEM((CHUNK, 128), _f32),
        pltpu.VMEM((RPT,), _f32),
        pltpu.VMEM_SHARED((NP, 128), _f32),
    ],
    compiler_params=_sc_compiler_params(),
)
def _deg_kernel(dstr, ewr, out, dst_v, ew_v, vals_v, deg1d_v, acc):
    c = lax.axis_index("c")
    s = lax.axis_index("s")

    @pl.loop(0, CHUNK)
    def _(r):
        for k in range(8):
            vals_v[r, pl.ds(k * 16, 16)] = jnp.zeros((16,), _f32)

    for m in range(RPT // CHUNK):
        pltpu.sync_copy(vals_v, acc.at[pl.ds(s * RPT + m * CHUNK, CHUNK)])
    plsc.subcore_barrier()

    @pl.loop(0, NCH // 16)
    def _(g0):
        g = g0 + c * (NCH // 16)
        pltpu.sync_copy(dstr.at[s, pl.ds(g * 8, 8)], dst_v)
        pltpu.sync_copy(ewr.at[s, pl.ds(g * 8, 8)], ew_v)

        for j in range(8):
            for k in range(CHUNK // 16):
                sl = pl.ds(k * 16, 16)
                ridx = lax.iota(_i32, 16) + (k * 16)
                plsc.store_scatter(vals_v, [ridx, jnp.zeros((16,), _i32)],
                                   ew_v[j, sl])
            pltpu.sync_copy(vals_v, acc.at[dst_v.at[j]], add=True)

    plsc.subcore_barrier()

    for m in range(RPT // CHUNK):
        pltpu.sync_copy(acc.at[pl.ds(s * RPT + m * CHUNK, CHUNK)], vals_v)

        @pl.loop(0, CHUNK // 16)
        def _(b):
            ridx = lax.iota(_i32, 16) + b * 16
            v = plsc.load_gather(vals_v, [ridx, jnp.zeros((16,), _i32)])
            deg1d_v[pl.ds(m * CHUNK + b * 16, 16)] = v

    pltpu.sync_copy(deg1d_v, out.at[pl.ds(c * NP + s * RPT, RPT)])


@functools.partial(
    pl.kernel,
    out_type=jax.ShapeDtypeStruct((2 * NP, 128), _f32),
    mesh=_MESH,
    scratch_types=[
        pltpu.VMEM((8, CHUNK), _i32),
        pltpu.VMEM((8, CHUNK), _i32),
        pltpu.VMEM((8, CHUNK), _f32),
        pltpu.VMEM((NP,), _f32),
        pltpu.VMEM((CHUNK,), _f32),
        pltpu.VMEM((CHUNK, 128), _f32),
        pltpu.VMEM((CHUNK, 128), _f32),
        pltpu.SemaphoreType.DMA,
        pltpu.SemaphoreType.DMA,
        pltpu.SemaphoreType.DMA,
        pltpu.SemaphoreType.DMA,
        pltpu.VMEM_SHARED((NP, 128), _f32),
    ],
    compiler_params=_sc_compiler_params(),
)
def _conv_kernel(t_cat, srcr, dstr, ewr, disr, out, src_v, dst_v, ew_v,
                 dis_v, sval_v, rows_a, rows_b, sga, sgb, ssa, ssb, acc):
    c = lax.axis_index("c")
    s = lax.axis_index("s")
    pltpu.sync_copy(disr, dis_v)

    @pl.loop(0, CHUNK)
    def _(r):
        for k in range(8):
            rows_a[r, pl.ds(k * 16, 16)] = jnp.zeros((16,), _f32)

    for m in range(RPT // CHUNK):
        pltpu.sync_copy(rows_a, acc.at[pl.ds(s * RPT + m * CHUNK, CHUNK)])
    plsc.subcore_barrier()

    bufs = ((rows_a, sga, ssa), (rows_b, sgb, ssb))

    @pl.loop(0, NCH // 8)
    def _(g):
        pltpu.sync_copy(srcr.at[c, s, pl.ds(g * 8, 8)], src_v)
        pltpu.sync_copy(dstr.at[s, pl.ds(g * 8, 8)], dst_v)
        pltpu.sync_copy(ewr.at[s, pl.ds(g * 8, 8)], ew_v)

        gathers = [None] * 8
        scatters = [None] * 8
        gathers[0] = pltpu.async_copy(t_cat.at[src_v.at[0]], rows_a, sga)
        for j in range(8):
            rows_p, _, sem_sp = bufs[j % 2]
            rows_q, sem_gq, sem_sq = bufs[(j + 1) % 2]

            for k in range(CHUNK // 16):
                sl = pl.ds(k * 16, 16)
                srck = src_v[j, sl] - c * N
                dsk = plsc.load_gather(dis_v, [srck])
                sval_v[sl] = ew_v[j, sl] * dsk

            gathers[j].wait()
            if j < 7:
                if j >= 1:
                    scatters[j - 1].wait()
                gathers[j + 1] = pltpu.async_copy(
                    t_cat.at[src_v.at[j + 1]], rows_q, sem_gq)

            @pl.loop(0, CHUNK)
            def _(r):
                sv = plsc.load_gather(sval_v, [jnp.full((16,), r, _i32)])
                for k in range(8):
                    sl = pl.ds(k * 16, 16)
                    rows_p[r, sl] = rows_p[r, sl] * sv

            scatters[j] = pltpu.async_copy(rows_p, acc.at[dst_v.at[j]],
                                           sem_sp, add=True)
        scatters[6].wait()
        scatters[7].wait()

    plsc.subcore_barrier()
    pltpu.sync_copy(acc.at[pl.ds(s * RPT, RPT)],
                    out.at[pl.ds(c * NP + s * RPT, RPT)])


_BN = 1000
_BE = 4000


def _dot(a, b):
    return jnp.dot(a, b, preferred_element_type=_f32)


def _edge_mlp_body(ef, w1, b1, w2, b2, out):
    hh = jnp.maximum(_dot(ef[...], w1[...]) + b1[...], 0.0)
    out[...] = _dot(hh, w2[...]) + b2[...]


def _edge_mlp(ef, w1, b1, w2, b2):
    return pl.pallas_call(
        _edge_mlp_body,
        grid=(E // _BE,),
        in_specs=[
            pl.BlockSpec((_BE, 16), lambda i: (i, 0)),
            pl.BlockSpec((16, 64), lambda i: (0, 0)),
            pl.BlockSpec((1, 64), lambda i: (0, 0)),
            pl.BlockSpec((64, 1), lambda i: (0, 0)),
            pl.BlockSpec((1, 1), lambda i: (0, 0)),
        ],
        out_specs=pl.BlockSpec((_BE, 1), lambda i: (i, 0)),
        out_shape=jax.ShapeDtypeStruct((E, 1), _f32),
    )(ef, w1, b1, w2, b2)


def _node_mlp_body(x, wn1, bn1, wn2, bn2, wg1, ta, tb):
    hh = jnp.maximum(_dot(x[...], wn1[...]) + bn1[...], 0.0)
    hh = _dot(hh, wn2[...]) + bn2[...]
    t = _dot(hh, wg1[...])
    ta[...] = t[:, :128]
    tb[...] = t[:, 128:]


def _node_mlp(x, wn1, bn1, wn2, bn2, wg1):
    return pl.pallas_call(
        _node_mlp_body,
        grid=(N // _BN,),
        in_specs=[
            pl.BlockSpec((_BN, 128), lambda i: (i, 0)),
            pl.BlockSpec((128, 256), lambda i: (0, 0)),
            pl.BlockSpec((1, 256), lambda i: (0, 0)),
            pl.BlockSpec((256, 256), lambda i: (0, 0)),
            pl.BlockSpec((1, 256), lambda i: (0, 0)),
            pl.BlockSpec((256, 256), lambda i: (0, 0)),
        ],
        out_specs=(pl.BlockSpec((_BN, 128), lambda i: (i, 0)),
                   pl.BlockSpec((_BN, 128), lambda i: (i, 0))),
        out_shape=(jax.ShapeDtypeStruct((N, 128), _f32),
                   jax.ShapeDtypeStruct((N, 128), _f32)),
    )(x, wn1, bn1, wn2, bn2, wg1)


def _dis_body(degs, dis):
    a = degs[...]
    deg = a[0:1, :] + a[1:2, :] + 1.0
    dis[...] = jnp.where(deg > 0, lax.rsqrt(jnp.maximum(deg, 1e-12)), 0.0)


def _dis_kernel(degs):
    return pl.pallas_call(
        _dis_body,
        out_shape=jax.ShapeDtypeStruct((1, NP), _f32),
    )(degs)


def _mid_body(agg_a, agg_b, ta, tb, dis, bg1, wg2, oa, ob):
    d = dis[...]
    agg = jnp.concatenate([agg_a[...], agg_b[...]], axis=1)
    t1 = jnp.concatenate([ta[...], tb[...]], axis=1)
    x1 = jnp.maximum(d * (agg + d * t1) + bg1[...], 0.0)
    t2 = _dot(x1, wg2[...])
    oa[...] = t2[:, :128]
    ob[...] = t2[:, 128:]


def _mid_kernel(agg_a, agg_b, ta, tb, dis, bg1, wg2):
    return pl.pallas_call(
        _mid_body,
        grid=(N // _BN,),
        in_specs=[
            pl.BlockSpec((_BN, 128), lambda i: (i, 0)),
            pl.BlockSpec((_BN, 128), lambda i: (i, 0)),
            pl.BlockSpec((_BN, 128), lambda i: (i, 0)),
            pl.BlockSpec((_BN, 128), lambda i: (i, 0)),
            pl.BlockSpec((_BN, 1), lambda i: (i, 0)),
            pl.BlockSpec((1, 256), lambda i: (0, 0)),
            pl.BlockSpec((256, 256), lambda i: (0, 0)),
        ],
        out_specs=(pl.BlockSpec((_BN, 128), lambda i: (i, 0)),
                   pl.BlockSpec((_BN, 128), lambda i: (i, 0))),
        out_shape=(jax.ShapeDtypeStruct((N, 128), _f32),
                   jax.ShapeDtypeStruct((N, 128), _f32)),
    )(agg_a, agg_b, ta, tb, dis, bg1, wg2)


def _final_body(agg_a, agg_b, ta, tb, dis, bg2, wo, bo, out):
    d = dis[...]
    agg = jnp.concatenate([agg_a[...], agg_b[...]], axis=1)
    t2 = jnp.concatenate([ta[...], tb[...]], axis=1)
    x2 = jnp.maximum(d * (agg + d * t2) + bg2[...], 0.0)
    out[...] = _dot(x2, wo[...]) + bo[...]


def _final_kernel(agg_a, agg_b, ta, tb, dis, bg2, wo, bo):
    return pl.pallas_call(
        _final_body,
        grid=(N // _BN,),
        in_specs=[
            pl.BlockSpec((_BN, 128), lambda i: (i, 0)),
            pl.BlockSpec((_BN, 128), lambda i: (i, 0)),
            pl.BlockSpec((_BN, 128), lambda i: (i, 0)),
            pl.BlockSpec((_BN, 128), lambda i: (i, 0)),
            pl.BlockSpec((_BN, 1), lambda i: (i, 0)),
            pl.BlockSpec((1, 256), lambda i: (0, 0)),
            pl.BlockSpec((256, 3), lambda i: (0, 0)),
            pl.BlockSpec((1, 3), lambda i: (0, 0)),
        ],
        out_specs=pl.BlockSpec((_BN, 3), lambda i: (i, 0)),
        out_shape=jax.ShapeDtypeStruct((N, 3), _f32),
    )(agg_a, agg_b, ta, tb, dis, bg2, wo, bo)


def kernel(node_features, edge_index, edge_features,
           W_n1, b_n1, W_n2, b_n2, W_e1, b_e1, W_e2, b_e2,
           W_g1, b_g1, W_g2, b_g2, W_o, b_o):
    src = edge_index[0].astype(_i32)
    dst = edge_index[1].astype(_i32)

    ew = _edge_mlp(edge_features, W_e1, b_e1.reshape(1, 64),
                   W_e2, b_e2.reshape(1, 1)).reshape(E)

    padn = EP - E
    fill = jnp.arange(padn, dtype=_i32)
    src_p = jnp.concatenate([src, fill % N]).reshape(NS, NCH, CHUNK)
    src_2 = jnp.stack([src_p, src_p + N])
    dst_p = jnp.concatenate([dst, fill % NP]).reshape(NS, NCH, CHUNK)
    ew_p = jnp.concatenate([ew, jnp.zeros((padn,), _f32)]).reshape(NS, NCH, CHUNK)

    t1a, t1b = _node_mlp(node_features, W_n1, b_n1.reshape(1, 256),
                         W_n2, b_n2.reshape(1, 256), W_g1)

    degs = _deg_kernel(dst_p, ew_p)
    dis_row = _dis_kernel(degs.reshape(2, NP))
    dis_flat = dis_row.reshape(NP)
    dis_col = dis_row.reshape(NP, 1)

    agg1 = _conv_kernel(jnp.concatenate([t1a, t1b], 0), src_2, dst_p, ew_p,
                        dis_flat)
    t2a, t2b = _mid_kernel(agg1[:N], agg1[NP:NP + N], t1a, t1b, dis_col[:N],
                           b_g1.reshape(1, 256), W_g2)

    agg2 = _conv_kernel(jnp.concatenate([t2a, t2b], 0), src_2, dst_p, ew_p,
                        dis_flat)
    return _final_kernel(agg2[:N], agg2[NP:NP + N], t2a, t2b, dis_col[:N],
                         b_g2.reshape(1, 256), W_o, b_o.reshape(1, 3))


# speedup vs baseline: 12.0234x; 1.1877x over previous
import dataclasses
import functools

import jax
import jax.numpy as jnp
from jax import lax
from jax.experimental import pallas as pl
from jax.experimental.pallas import tpu as pltpu
from jax.experimental.pallas import tpu_sc as plsc

N = 10000
E = 320000
NP = 10240
NS = 16
CHUNK = 128
EPT = 20480
EP = EPT * NS
NCH = EPT // CHUNK
RPT = NP // NS

_f32 = jnp.float32
_i32 = jnp.int32


def _sc_compiler_params():
    cp = pltpu.CompilerParams()
    if "needs_layout_passes" in pltpu.CompilerParams.__dataclass_fields__:
        cp = dataclasses.replace(cp, needs_layout_passes=False)
    return cp


_MESH = plsc.VectorSubcoreMesh(core_axis_name="c", subcore_axis_name="s")


@functools.partial(
    pl.kernel,
    out_type=jax.ShapeDtypeStruct((2 * NP,), _f32),
    mesh=_MESH,
    scratch_types=[
        pltpu.VMEM((8, CHUNK), _i32),
        pltpu.VMEM((8, CHUNK), _f32),
        pltpu.VMEM((CHUNK, 128), _f32),
        pltpu.VMEM((RPT,), _f32),
        pltpu.VMEM_SHARED((NP, 128), _f32),
    ],
    compiler_params=_sc_compiler_params(),
)
def _deg_kernel(dstr, ewr, out, dst_v, ew_v, vals_v, deg1d_v, acc):
    c = lax.axis_index("c")
    s = lax.axis_index("s")

    @pl.loop(0, CHUNK)
    def _(r):
        for k in range(8):
            vals_v[r, pl.ds(k * 16, 16)] = jnp.zeros((16,), _f32)

    for m in range(RPT // CHUNK):
        pltpu.sync_copy(vals_v, acc.at[pl.ds(s * RPT + m * CHUNK, CHUNK)])
    plsc.subcore_barrier()

    @pl.loop(0, NCH // 16)
    def _(g0):
        g = g0 + c * (NCH // 16)
        pltpu.sync_copy(dstr.at[s, pl.ds(g * 8, 8)], dst_v)
        pltpu.sync_copy(ewr.at[s, pl.ds(g * 8, 8)], ew_v)

        for j in range(8):
            for k in range(CHUNK // 16):
                sl = pl.ds(k * 16, 16)
                ridx = lax.iota(_i32, 16) + (k * 16)
                plsc.store_scatter(vals_v, [ridx, jnp.zeros((16,), _i32)],
                                   ew_v[j, sl])
            pltpu.sync_copy(vals_v, acc.at[dst_v.at[j]], add=True)

    plsc.subcore_barrier()

    for m in range(RPT // CHUNK):
        pltpu.sync_copy(acc.at[pl.ds(s * RPT + m * CHUNK, CHUNK)], vals_v)

        @pl.loop(0, CHUNK // 16)
        def _(b):
            ridx = lax.iota(_i32, 16) + b * 16
            v = plsc.load_gather(vals_v, [ridx, jnp.zeros((16,), _i32)])
            deg1d_v[pl.ds(m * CHUNK + b * 16, 16)] = v

    pltpu.sync_copy(deg1d_v, out.at[pl.ds(c * NP + s * RPT, RPT)])


@functools.partial(
    pl.kernel,
    out_type=jax.ShapeDtypeStruct((2 * NP, 128), _f32),
    mesh=_MESH,
    scratch_types=[
        pltpu.VMEM((8, CHUNK), _i32),
        pltpu.VMEM((8, CHUNK), _i32),
        pltpu.VMEM((8, CHUNK), _f32),
        pltpu.VMEM((NP,), _f32),
        pltpu.VMEM((CHUNK,), _f32),
        pltpu.VMEM((CHUNK, 128), _f32),
        pltpu.VMEM((CHUNK, 128), _f32),
        pltpu.SemaphoreType.DMA,
        pltpu.SemaphoreType.DMA,
        pltpu.SemaphoreType.DMA,
        pltpu.SemaphoreType.DMA,
        pltpu.VMEM_SHARED((NP, 128), _f32),
    ],
    compiler_params=_sc_compiler_params(),
)
def _conv_kernel(t_cat, srcr, dstr, ewr, disr, out, src_v, dst_v, ew_v,
                 dis_v, sval_v, rows_a, rows_b, sga, sgb, ssa, ssb, acc):
    c = lax.axis_index("c")
    s = lax.axis_index("s")
    pltpu.sync_copy(disr, dis_v)

    @pl.loop(0, CHUNK)
    def _(r):
        for k in range(8):
            rows_a[r, pl.ds(k * 16, 16)] = jnp.zeros((16,), _f32)

    for m in range(RPT // CHUNK):
        pltpu.sync_copy(rows_a, acc.at[pl.ds(s * RPT + m * CHUNK, CHUNK)])
    plsc.subcore_barrier()

    bufs = ((rows_a, sga, ssa), (rows_b, sgb, ssb))

    @pl.loop(0, NCH // 8)
    def _(g):
        pltpu.sync_copy(srcr.at[c, s, pl.ds(g * 8, 8)], src_v)
        pltpu.sync_copy(dstr.at[s, pl.ds(g * 8, 8)], dst_v)
        pltpu.sync_copy(ewr.at[s, pl.ds(g * 8, 8)], ew_v)

        gathers = [None] * 8
        scatters = [None] * 8
        gathers[0] = pltpu.async_copy(t_cat.at[src_v.at[0]], rows_a, sga)
        for j in range(8):
            rows_p, _, sem_sp = bufs[j % 2]
            rows_q, sem_gq, sem_sq = bufs[(j + 1) % 2]

            for k in range(CHUNK // 16):
                sl = pl.ds(k * 16, 16)
                srck = src_v[j, sl] - c * N
                dsk = plsc.load_gather(dis_v, [srck])
                sval_v[sl] = ew_v[j, sl] * dsk

            gathers[j].wait()
            if j < 7:
                if j >= 1:
                    scatters[j - 1].wait()
                gathers[j + 1] = pltpu.async_copy(
                    t_cat.at[src_v.at[j + 1]], rows_q, sem_gq)

            @functools.partial(plsc.parallel_loop, 0, CHUNK, unroll=4)
            def _(r):
                sv = plsc.load_gather(sval_v, [jnp.full((16,), r, _i32)])
                for k in range(8):
                    sl = pl.ds(k * 16, 16)
                    rows_p[r, sl] = rows_p[r, sl] * sv

            scatters[j] = pltpu.async_copy(rows_p, acc.at[dst_v.at[j]],
                                           sem_sp, add=True)
        scatters[6].wait()
        scatters[7].wait()

    plsc.subcore_barrier()
    pltpu.sync_copy(acc.at[pl.ds(s * RPT, RPT)],
                    out.at[pl.ds(c * NP + s * RPT, RPT)])


_BN = 1000
_BE = 4000


def _dot(a, b):
    return jnp.dot(a, b, preferred_element_type=_f32)


def _edge_mlp_body(ef, w1, b1, w2, b2, out):
    hh = jnp.maximum(_dot(ef[...], w1[...]) + b1[...], 0.0)
    out[...] = _dot(hh, w2[...]) + b2[...]


def _edge_mlp(ef, w1, b1, w2, b2):
    return pl.pallas_call(
        _edge_mlp_body,
        grid=(E // _BE,),
        in_specs=[
            pl.BlockSpec((_BE, 16), lambda i: (i, 0)),
            pl.BlockSpec((16, 64), lambda i: (0, 0)),
            pl.BlockSpec((1, 64), lambda i: (0, 0)),
            pl.BlockSpec((64, 1), lambda i: (0, 0)),
            pl.BlockSpec((1, 1), lambda i: (0, 0)),
        ],
        out_specs=pl.BlockSpec((_BE, 1), lambda i: (i, 0)),
        out_shape=jax.ShapeDtypeStruct((E, 1), _f32),
    )(ef, w1, b1, w2, b2)


def _node_mlp_body(x, wn1, bn1, wn2, bn2, wg1, ta, tb):
    hh = jnp.maximum(_dot(x[...], wn1[...]) + bn1[...], 0.0)
    hh = _dot(hh, wn2[...]) + bn2[...]
    t = _dot(hh, wg1[...])
    ta[...] = t[:, :128]
    tb[...] = t[:, 128:]


def _node_mlp(x, wn1, bn1, wn2, bn2, wg1):
    return pl.pallas_call(
        _node_mlp_body,
        grid=(N // _BN,),
        in_specs=[
            pl.BlockSpec((_BN, 128), lambda i: (i, 0)),
            pl.BlockSpec((128, 256), lambda i: (0, 0)),
            pl.BlockSpec((1, 256), lambda i: (0, 0)),
            pl.BlockSpec((256, 256), lambda i: (0, 0)),
            pl.BlockSpec((1, 256), lambda i: (0, 0)),
            pl.BlockSpec((256, 256), lambda i: (0, 0)),
        ],
        out_specs=(pl.BlockSpec((_BN, 128), lambda i: (i, 0)),
                   pl.BlockSpec((_BN, 128), lambda i: (i, 0))),
        out_shape=(jax.ShapeDtypeStruct((N, 128), _f32),
                   jax.ShapeDtypeStruct((N, 128), _f32)),
    )(x, wn1, bn1, wn2, bn2, wg1)


def _dis_body(degs, dis):
    a = degs[...]
    deg = a[0:1, :] + a[1:2, :] + 1.0
    dis[...] = jnp.where(deg > 0, lax.rsqrt(jnp.maximum(deg, 1e-12)), 0.0)


def _dis_kernel(degs):
    return pl.pallas_call(
        _dis_body,
        out_shape=jax.ShapeDtypeStruct((1, NP), _f32),
    )(degs)


def _mid_body(agg_a, agg_b, ta, tb, dis, bg1, wg2, oa, ob):
    d = dis[...]
    agg = jnp.concatenate([agg_a[...], agg_b[...]], axis=1)
    t1 = jnp.concatenate([ta[...], tb[...]], axis=1)
    x1 = jnp.maximum(d * (agg + d * t1) + bg1[...], 0.0)
    t2 = _dot(x1, wg2[...])
    oa[...] = t2[:, :128]
    ob[...] = t2[:, 128:]


def _mid_kernel(agg_a, agg_b, ta, tb, dis, bg1, wg2):
    return pl.pallas_call(
        _mid_body,
        grid=(N // _BN,),
        in_specs=[
            pl.BlockSpec((_BN, 128), lambda i: (i, 0)),
            pl.BlockSpec((_BN, 128), lambda i: (i, 0)),
            pl.BlockSpec((_BN, 128), lambda i: (i, 0)),
            pl.BlockSpec((_BN, 128), lambda i: (i, 0)),
            pl.BlockSpec((_BN, 1), lambda i: (i, 0)),
            pl.BlockSpec((1, 256), lambda i: (0, 0)),
            pl.BlockSpec((256, 256), lambda i: (0, 0)),
        ],
        out_specs=(pl.BlockSpec((_BN, 128), lambda i: (i, 0)),
                   pl.BlockSpec((_BN, 128), lambda i: (i, 0))),
        out_shape=(jax.ShapeDtypeStruct((N, 128), _f32),
                   jax.ShapeDtypeStruct((N, 128), _f32)),
    )(agg_a, agg_b, ta, tb, dis, bg1, wg2)


def _final_body(agg_a, agg_b, ta, tb, dis, bg2, wo, bo, out):
    d = dis[...]
    agg = jnp.concatenate([agg_a[...], agg_b[...]], axis=1)
    t2 = jnp.concatenate([ta[...], tb[...]], axis=1)
    x2 = jnp.maximum(d * (agg + d * t2) + bg2[...], 0.0)
    out[...] = _dot(x2, wo[...]) + bo[...]


def _final_kernel(agg_a, agg_b, ta, tb, dis, bg2, wo, bo):
    return pl.pallas_call(
        _final_body,
        grid=(N // _BN,),
        in_specs=[
            pl.BlockSpec((_BN, 128), lambda i: (i, 0)),
            pl.BlockSpec((_BN, 128), lambda i: (i, 0)),
            pl.BlockSpec((_BN, 128), lambda i: (i, 0)),
            pl.BlockSpec((_BN, 128), lambda i: (i, 0)),
            pl.BlockSpec((_BN, 1), lambda i: (i, 0)),
            pl.BlockSpec((1, 256), lambda i: (0, 0)),
            pl.BlockSpec((256, 3), lambda i: (0, 0)),
            pl.BlockSpec((1, 3), lambda i: (0, 0)),
        ],
        out_specs=pl.BlockSpec((_BN, 3), lambda i: (i, 0)),
        out_shape=jax.ShapeDtypeStruct((N, 3), _f32),
    )(agg_a, agg_b, ta, tb, dis, bg2, wo, bo)


def kernel(node_features, edge_index, edge_features,
           W_n1, b_n1, W_n2, b_n2, W_e1, b_e1, W_e2, b_e2,
           W_g1, b_g1, W_g2, b_g2, W_o, b_o):
    src = edge_index[0].astype(_i32)
    dst = edge_index[1].astype(_i32)

    ew = _edge_mlp(edge_features, W_e1, b_e1.reshape(1, 64),
                   W_e2, b_e2.reshape(1, 1)).reshape(E)

    padn = EP - E
    fill = jnp.arange(padn, dtype=_i32)
    src_p = jnp.concatenate([src, fill % N]).reshape(NS, NCH, CHUNK)
    src_2 = jnp.stack([src_p, src_p + N])
    dst_p = jnp.concatenate([dst, fill % NP]).reshape(NS, NCH, CHUNK)
    ew_p = jnp.concatenate([ew, jnp.zeros((padn,), _f32)]).reshape(NS, NCH, CHUNK)

    t1a, t1b = _node_mlp(node_features, W_n1, b_n1.reshape(1, 256),
                         W_n2, b_n2.reshape(1, 256), W_g1)

    degs = _deg_kernel(dst_p, ew_p)
    dis_row = _dis_kernel(degs.reshape(2, NP))
    dis_flat = dis_row.reshape(NP)
    dis_col = dis_row.reshape(NP, 1)

    agg1 = _conv_kernel(jnp.concatenate([t1a, t1b], 0), src_2, dst_p, ew_p,
                        dis_flat)
    t2a, t2b = _mid_kernel(agg1[:N], agg1[NP:NP + N], t1a, t1b, dis_col[:N],
                           b_g1.reshape(1, 256), W_g2)

    agg2 = _conv_kernel(jnp.concatenate([t2a, t2b], 0), src_2, dst_p, ew_p,
                        dis_flat)
    return _final_kernel(agg2[:N], agg2[NP:NP + N], t2a, t2b, dis_col[:N],
                         b_g2.reshape(1, 256), W_o, b_o.reshape(1, 3))
